# Initial kernel scaffold; baseline (speedup 1.0000x reference)
import jax
import jax.numpy as jnp
from jax.experimental import pallas as pl


def kernel(x_a, edge_index_a, batch_a, x_b, edge_index_b, batch_b, labels, W1l, b1l, W1r, W2l, b2l, W2r, W3l, b3l, W3r, Wqp, bqp, Wkp, bkp, Wvp, bvp, Wqi, bqi, Wki, bki, Wvi, bvi, Wo, bo, ln_g, ln_b, Wd1, bd1, Wd2, bd2):
    raise NotImplementedError("write your pallas kernel here")


# speedup vs baseline: 2.3598x; 2.3598x over previous
import functools

import numpy as np
import jax
import jax.numpy as jnp
from jax import lax
from jax.experimental import pallas as pl
from jax.experimental.pallas import tpu as pltpu
from jax.experimental.pallas import tpu_sc as plsc

N = 10000
E = 160000
D = 256
HID = 256
B = 64
BB_RATIO = 0.1
ATT_RATIO = 0.15
ALPHA = 1.0
BETA = 1.0
MARGIN = 1.0

NC = 2
NS = 16
HD = 128
NP = 10240
RPT = NP // NS

EC = 80
EPT = E // NS
NCHUNK = EPT // EC

EPAD = 163840
DTILE = EPAD // (NC * NS)
DC = 80
NDCHUNK = DTILE // DC
HN = 10016

BLK = 1000
GRID = N // BLK

_F32 = jnp.float32


def _make_masks():
    mk = jax.random.key(42)
    mka, mkb = jax.random.split(mk)
    out = []
    for mkey in (mka, mkb):
        k1, k2 = jax.random.split(mkey)
        keep_bb = jax.random.bernoulli(k1, 1.0 - BB_RATIO, (N, D))
        keep_att = jax.random.bernoulli(k2, 1.0 - ATT_RATIO, (N, D))
        kb = keep_bb.astype(jnp.float32)
        mi = (~keep_att).astype(jnp.float32)
        out.append((kb, mi))
    return out

def _zero_rows(ref, nrows):
    def body(i, _):
        for j in range(HD // 16):
            ref[i, pl.ds(j * 16, 16)] = jnp.zeros((16,), _F32)
        return 0
    lax.fori_loop(0, nrows, body, 0)


def _sc_agg_body(htab, src, dst, agg_out, src_v, dst_v, rows_v, acc, sem):
    c = lax.axis_index("c")
    s = lax.axis_index("s")
    coff = c * N

    _zero_rows(rows_v, EC)
    for k in range(RPT // EC):
        pltpu.sync_copy(rows_v, acc.at[pl.ds(s * RPT + k * EC, EC)])
    plsc.subcore_barrier()

    def chunk(j, _):
        base = s * EPT + j * EC
        pltpu.sync_copy(src.at[pl.ds(base, EC)], src_v)
        pltpu.sync_copy(dst.at[pl.ds(base, EC)], dst_v)
        for t in range(EC // 16):
            sl = pl.ds(t * 16, 16)
            src_v[sl] = src_v[sl] + coff
        pltpu.async_copy(htab.at[src_v], rows_v, sem).wait()
        pltpu.sync_copy(rows_v, acc.at[dst_v], add=True)
        return 0
    lax.fori_loop(0, NCHUNK, chunk, 0)

    plsc.subcore_barrier()

    for k in range(RPT // EC):
        r0 = s * RPT + k * EC
        pltpu.sync_copy(acc.at[pl.ds(r0, EC)],
                        agg_out.at[pl.ds(c * NP + r0, EC)])


def _sc_deg_body(dstp, out, dst_v, hist):
    c = lax.axis_index("c")
    s = lax.axis_index("s")
    wid = c * NS + s

    def zb(i, _):
        hist[pl.ds(i * 16, 16)] = jnp.zeros((16,), _F32)
        return 0
    lax.fori_loop(0, HN // 16, zb, 0)

    ones16 = jnp.ones((16,), _F32)

    def chunk(j, _):
        base = wid * DTILE + j * DC
        pltpu.sync_copy(dstp.at[pl.ds(base, DC)], dst_v)
        for t in range(DC // 16):
            idx = dst_v[pl.ds(t * 16, 16)]
            plsc.addupdate_scatter(hist, [idx], ones16)
        return 0
    lax.fori_loop(0, NDCHUNK, chunk, 0)

    pltpu.sync_copy(hist, out.at[pl.ds(wid * HN, HN)])


@functools.lru_cache(maxsize=None)
def _build_sc_kernels():
    mesh = plsc.VectorSubcoreMesh(core_axis_name="c", subcore_axis_name="s")
    agg = pl.kernel(
        _sc_agg_body,
        out_type=jax.ShapeDtypeStruct((NC * NP, HD), _F32),
        mesh=mesh,
        scratch_types=(
            pltpu.VMEM((EC,), jnp.int32),
            pltpu.VMEM((EC,), jnp.int32),
            pltpu.VMEM((EC, HD), _F32),
            pltpu.VMEM_SHARED((NP, HD), _F32),
            pltpu.SemaphoreType.DMA,
        ),
    )
    deg = pl.kernel(
        _sc_deg_body,
        out_type=jax.ShapeDtypeStruct((NC * NS * HN,), _F32),
        mesh=mesh,
        compiler_params=pltpu.CompilerParams(needs_layout_passes=False),
        scratch_types=(
            pltpu.VMEM((DC,), jnp.int32),
            pltpu.VMEM((HN,), _F32),
        ),
    )
    return agg, deg


def _sc_agg(hpair, src, dst):
    agg = _build_sc_kernels()[0](hpair.reshape(NC * N, HD), src, dst)
    return agg.reshape(NC, NP, HD)


def _sc_deg(dst):
    dstp = jnp.concatenate(
        [dst, jnp.full((EPAD - E,), N, jnp.int32)])
    out = _build_sc_kernels()[1](dstp)
    return out.reshape(NC * NS, HN)[:, :N].T


def _mask_body(x_ref, kb_ref, out_ref):
    xb = x_ref[...] * kb_ref[...]
    out_ref[0] = xb[:, :HD]
    out_ref[1] = xb[:, HD:]


def _mask_mul(x, kb):
    return pl.pallas_call(
        _mask_body,
        grid=(GRID,),
        in_specs=[pl.BlockSpec((BLK, D), lambda i: (i, 0)),
                  pl.BlockSpec((BLK, D), lambda i: (i, 0))],
        out_specs=pl.BlockSpec((NC, BLK, HD), lambda i: (0, i, 0)),
        out_shape=jax.ShapeDtypeStruct((NC, N, HD), _F32),
    )(x, kb)


def _dense_body(relu, agg_ref, deg_ref, h_ref, wl_ref, bl_ref, wr_ref,
                out_ref):
    aggc = jnp.concatenate([agg_ref[0], agg_ref[1]], axis=1)
    deg = jnp.sum(deg_ref[...], axis=1)[:, None]
    inv = 1.0 / jnp.maximum(deg, 1.0)
    hc = jnp.concatenate([h_ref[0], h_ref[1]], axis=1)
    out = (jnp.dot(aggc * inv, wl_ref[...].T,
                   preferred_element_type=_F32)
           + bl_ref[...]
           + jnp.dot(hc, wr_ref[...].T, preferred_element_type=_F32))
    if relu:
        out = jnp.maximum(out, 0.0)
    out_ref[0] = out[:, :HD]
    out_ref[1] = out[:, HD:]


def _dense(agg, deg, h, wl, bl, wr, relu):
    return pl.pallas_call(
        functools.partial(_dense_body, relu),
        grid=(GRID,),
        in_specs=[pl.BlockSpec((NC, BLK, HD), lambda i: (0, i, 0)),
                  pl.BlockSpec((BLK, NC * NS), lambda i: (i, 0)),
                  pl.BlockSpec((NC, BLK, HD), lambda i: (0, i, 0)),
                  pl.BlockSpec((HID, HID), lambda i: (0, 0)),
                  pl.BlockSpec((1, HID), lambda i: (0, 0)),
                  pl.BlockSpec((HID, HID), lambda i: (0, 0))],
        out_specs=pl.BlockSpec((NC, BLK, HD), lambda i: (0, i, 0)),
        out_shape=jax.ShapeDtypeStruct((NC, N, HD), _F32),
    )(agg, deg, h, wl, bl, wr)


def _pool_body(agg_ref, deg_ref, h_ref, wl_ref, bl_ref, wr_ref, batch_ref,
               gs_ref, cs_ref):
    aggc = jnp.concatenate([agg_ref[0], agg_ref[1]], axis=1)
    deg = jnp.sum(deg_ref[...], axis=1)[:, None]
    inv = 1.0 / jnp.maximum(deg, 1.0)
    hc = jnp.concatenate([h_ref[0], h_ref[1]], axis=1)
    h3 = (jnp.dot(aggc * inv, wl_ref[...].T, preferred_element_type=_F32)
          + bl_ref[...]
          + jnp.dot(hc, wr_ref[...].T, preferred_element_type=_F32))
    bb = batch_ref[0, 0, :]
    gid = lax.broadcasted_iota(jnp.int32, (B, BLK), 0)
    onehot = (gid == bb[None, :]).astype(_F32)
    pool = jnp.dot(onehot, h3, preferred_element_type=_F32)
    cpool = jnp.dot(onehot, jnp.ones((BLK, 16), _F32),
                    preferred_element_type=_F32)

    @pl.when(pl.program_id(0) == 0)
    def _():
        gs_ref[...] = jnp.zeros_like(gs_ref)
        cs_ref[...] = jnp.zeros_like(cs_ref)

    gs_ref[...] += pool
    cs_ref[...] += cpool


def _pool(agg, deg, h, wl, bl, wr, batch3d):
    return pl.pallas_call(
        _pool_body,
        grid=(GRID,),
        in_specs=[pl.BlockSpec((NC, BLK, HD), lambda i: (0, i, 0)),
                  pl.BlockSpec((BLK, NC * NS), lambda i: (i, 0)),
                  pl.BlockSpec((NC, BLK, HD), lambda i: (0, i, 0)),
                  pl.BlockSpec((HID, HID), lambda i: (0, 0)),
                  pl.BlockSpec((1, HID), lambda i: (0, 0)),
                  pl.BlockSpec((HID, HID), lambda i: (0, 0)),
                  pl.BlockSpec((1, 1, BLK), lambda i: (i, 0, 0))],
        out_specs=[pl.BlockSpec((B, HID), lambda i: (0, 0)),
                   pl.BlockSpec((B, 16), lambda i: (0, 0))],
        out_shape=[jax.ShapeDtypeStruct((B, HID), _F32),
                   jax.ShapeDtypeStruct((B, 16), _F32)],
    )(agg, deg, h, wl, bl, wr, batch3d)


def _graph_chain(gs, cs, wvp, bvp, wvi, bvi, wo, bo, lng, lnb,
                 wd1, bd1, wd2, bd2):
    gvec = gs * (1.0 / jnp.maximum(cs[:, 0:1], 1.0))
    vv = jnp.dot(gvec, wvp.T, preferred_element_type=_F32) + bvp
    vi = jnp.dot(vv, wvi.T, preferred_element_type=_F32) + bvi
    att = jnp.dot(vi, wo.T, preferred_element_type=_F32) + bo
    m = jnp.mean(att, axis=-1, keepdims=True)
    cen = att - m
    var = jnp.mean(cen * cen, axis=-1, keepdims=True)
    ln = cen / jnp.sqrt(var + 1e-5) * lng + lnb
    d1 = jnp.maximum(jnp.dot(ln, wd1.T, preferred_element_type=_F32) + bd1,
                     0.0)
    pg = jnp.dot(d1, wd2.T, preferred_element_type=_F32) + bd2
    return gvec, pg


def _graph_body(gsa_ref, csa_ref, gsb_ref, csb_ref, lab_ref,
                wvp_ref, bvp_ref, wvi_ref, bvi_ref, wo_ref, bo_ref,
                lng_ref, lnb_ref, wd1_ref, bd1_ref, wd2_ref, bd2_ref,
                pga_ref, pgb_ref, lc_ref):
    args = (wvp_ref[...], bvp_ref[...], wvi_ref[...], bvi_ref[...],
            wo_ref[...], bo_ref[...], lng_ref[...], lnb_ref[...],
            wd1_ref[...], bd1_ref[...], wd2_ref[...], bd2_ref[...])
    gva, pga = _graph_chain(gsa_ref[...], csa_ref[...], *args)
    gvb, pgb = _graph_chain(gsb_ref[...], csb_ref[...], *args)
    pga_ref[...] = pga
    pgb_ref[...] = pgb
    diff = gva - gvb + 1e-6
    dist = jnp.sqrt(jnp.sum(diff * diff, axis=-1, keepdims=True))
    lab = lab_ref[...]
    hinge = jnp.maximum(MARGIN - dist, 0.0)
    lc = jnp.mean(lab * dist * dist + (1.0 - lab) * hinge * hinge)
    lc_ref[...] = jnp.full((1, 1), lc, _F32)


def _graph(gsa, csa, gsb, csb, lab2d, wvp, bvp, wvi, bvi, wo, bo,
           lng, lnb, wd1, bd1, wd2, bd2):
    return pl.pallas_call(
        _graph_body,
        out_shape=[jax.ShapeDtypeStruct((B, HID), _F32),
                   jax.ShapeDtypeStruct((B, HID), _F32),
                   jax.ShapeDtypeStruct((1, 1), _F32)],
    )(gsa, csa, gsb, csb, lab2d, wvp, bvp, wvi, bvi, wo, bo,
      lng, lnb, wd1, bd1, wd2, bd2)


def _lpred_body(x_ref, mi_ref, batch_ref, pg_ref, out_ref, cnt_ref):
    bb = batch_ref[0, 0, :]
    nid = lax.broadcasted_iota(jnp.int32, (BLK, B), 1)
    oh = (nid == bb[:, None]).astype(_F32)
    pgb = jnp.dot(oh, pg_ref[...], preferred_element_type=_F32)
    d = pgb - x_ref[...]
    mi = mi_ref[...]
    val = jnp.sum(d * d * mi)
    cval = jnp.sum(mi)

    @pl.when(pl.program_id(0) == 0)
    def _():
        out_ref[...] = jnp.zeros((1, 1), _F32)
        cnt_ref[...] = jnp.zeros((1, 1), _F32)

    out_ref[...] = out_ref[...] + val
    cnt_ref[...] = cnt_ref[...] + cval


def _lpred(x, mi, batch3d, pg):
    return pl.pallas_call(
        _lpred_body,
        grid=(GRID,),
        in_specs=[pl.BlockSpec((BLK, D), lambda i: (i, 0)),
                  pl.BlockSpec((BLK, D), lambda i: (i, 0)),
                  pl.BlockSpec((1, 1, BLK), lambda i: (i, 0, 0)),
                  pl.BlockSpec((B, HID), lambda i: (0, 0))],
        out_specs=[pl.BlockSpec((1, 1), lambda i: (0, 0)),
                   pl.BlockSpec((1, 1), lambda i: (0, 0))],
        out_shape=[jax.ShapeDtypeStruct((1, 1), _F32),
                   jax.ShapeDtypeStruct((1, 1), _F32)],
    )(x, mi, batch3d, pg)


def _branch_nodes(x, src, dst, batch3d, kb, W1l, b1l, W1r, W2l, b2l, W2r,
                  W3l, b3l, W3r):
    xbb = _mask_mul(x, kb)
    deg = _sc_deg(dst)
    agg1 = _sc_agg(xbb, src, dst)
    h1 = _dense(agg1, deg, xbb, W1l, b1l, W1r, relu=True)
    agg2 = _sc_agg(h1, src, dst)
    h2 = _dense(agg2, deg, h1, W2l, b2l, W2r, relu=True)
    agg3 = _sc_agg(h2, src, dst)
    gs, cs = _pool(agg3, deg, h2, W3l, b3l, W3r, batch3d)
    return gs, cs


def kernel(x_a, edge_index_a, batch_a, x_b, edge_index_b, batch_b, labels,
           W1l, b1l, W1r, W2l, b2l, W2r, W3l, b3l, W3r, Wqp, bqp, Wkp, bkp,
           Wvp, bvp, Wqi, bqi, Wki, bki, Wvi, bvi, Wo, bo, ln_g, ln_b,
           Wd1, bd1, Wd2, bd2):
    src_a = edge_index_a[0]
    dst_a = edge_index_a[1]
    src_b = edge_index_b[0]
    dst_b = edge_index_b[1]
    batch3d_a = batch_a.reshape(GRID, 1, BLK)
    batch3d_b = batch_b.reshape(GRID, 1, BLK)
    b1l2 = b1l.reshape(1, HID)
    b2l2 = b2l.reshape(1, HID)
    b3l2 = b3l.reshape(1, HID)

    (kba, mia), (kbb, mib) = _make_masks()
    gs_a, cs_a = _branch_nodes(x_a, src_a, dst_a, batch3d_a, kba,
                               W1l, b1l2, W1r, W2l, b2l2, W2r,
                               W3l, b3l2, W3r)
    gs_b, cs_b = _branch_nodes(x_b, src_b, dst_b, batch3d_b, kbb,
                               W1l, b1l2, W1r, W2l, b2l2, W2r,
                               W3l, b3l2, W3r)

    pga, pgb, lc = _graph(gs_a, cs_a, gs_b, cs_b, labels.reshape(B, 1),
                          Wvp, bvp.reshape(1, HID), Wvi, bvi.reshape(1, HID),
                          Wo, bo.reshape(1, HID), ln_g.reshape(1, HID),
                          ln_b.reshape(1, HID), Wd1, bd1.reshape(1, HID),
                          Wd2, bd2.reshape(1, HID))

    sa, ca = _lpred(x_a, mia, batch3d_a, pga)
    sb, cb = _lpred(x_b, mib, batch3d_b, pgb)

    loss_con = lc[0, 0]
    lpa = sa[0, 0] / jnp.maximum(ca[0, 0], 1.0)
    lpb = sb[0, 0] / jnp.maximum(cb[0, 0], 1.0)
    loss_pred = (lpa + lpb) * 0.5
    total = ALPHA * loss_con + BETA * loss_pred
    return (total, loss_con, loss_pred)


# speedup vs baseline: 2.4256x; 1.0279x over previous
import functools

import numpy as np
import jax
import jax.numpy as jnp
from jax import lax
from jax.experimental import pallas as pl
from jax.experimental.pallas import tpu as pltpu
from jax.experimental.pallas import tpu_sc as plsc

N = 10000
E = 160000
D = 256
HID = 256
B = 64
BB_RATIO = 0.1
ATT_RATIO = 0.15
ALPHA = 1.0
BETA = 1.0
MARGIN = 1.0

NC = 2
NS = 16
HD = 128
NP = 10240
RPT = NP // NS

EC = 128
EPAD = 163840
EPT = EPAD // NS
NCHUNK = EPT // EC
CPH = NCHUNK // 2

DTILE = EPAD // (NC * NS)
DC = 80
NDCHUNK = DTILE // DC
HN = NP

BLK = 1000
GRID = N // BLK

_F32 = jnp.float32


def _make_masks():
    mk = jax.random.key(42)
    mka, mkb = jax.random.split(mk)
    out = []
    for mkey in (mka, mkb):
        k1, k2 = jax.random.split(mkey)
        keep_bb = jax.random.bernoulli(k1, 1.0 - BB_RATIO, (N, D))
        keep_att = jax.random.bernoulli(k2, 1.0 - ATT_RATIO, (N, D))
        kb = keep_bb.astype(jnp.float32)
        mi = (~keep_att).astype(jnp.float32)
        out.append((kb, mi))
    return out

def _zero_rows(ref, nrows):
    def body(i, _):
        for j in range(HD // 16):
            ref[i, pl.ds(j * 16, 16)] = jnp.zeros((16,), _F32)
        return 0
    lax.fori_loop(0, nrows, body, 0)


def _sc_agg_body(htab, src3, dst3, agg_out, src2_v, dst2_v, rows0, rows1,
                 acc, sem_a, sem_b):
    c = lax.axis_index("c")
    s = lax.axis_index("s")
    coff = c * N

    _zero_rows(rows0, EC)
    for k in range(RPT // EC):
        pltpu.sync_copy(rows0, acc.at[pl.ds(s * RPT + k * EC, EC)])

    plsc.subcore_barrier()

    for ph in range(NCHUNK // CPH):
        pltpu.sync_copy(src3.at[s, pl.ds(ph * CPH, CPH)], src2_v)
        pltpu.sync_copy(dst3.at[s, pl.ds(ph * CPH, CPH)], dst2_v)

        def adj(i, _):
            for t in range(EC // 16):
                sl = pl.ds(t * 16, 16)
                src2_v[i, sl] = src2_v[i, sl] + coff
            return 0
        lax.fori_loop(0, CPH, adj, 0)

        pltpu.async_copy(htab.at[src2_v.at[0]], rows0, sem_a)

        def pair(k, _):
            j0 = 2 * k
            pltpu.make_async_copy(htab.at[src2_v.at[j0]], rows0,
                                  sem_a).wait()
            pltpu.async_copy(htab.at[src2_v.at[j0 + 1]], rows1, sem_b)
            pltpu.sync_copy(rows0, acc.at[dst2_v.at[j0]], add=True)
            pltpu.make_async_copy(htab.at[src2_v.at[j0 + 1]], rows1,
                                  sem_b).wait()

            @pl.when(k < CPH // 2 - 1)
            def _():
                pltpu.async_copy(htab.at[src2_v.at[j0 + 2]], rows0, sem_a)

            pltpu.sync_copy(rows1, acc.at[dst2_v.at[j0 + 1]], add=True)
            return 0
        lax.fori_loop(0, CPH // 2, pair, 0)

    plsc.subcore_barrier()

    for k in range(RPT // EC):
        r0 = s * RPT + k * EC
        pltpu.sync_copy(acc.at[pl.ds(r0, EC)],
                        agg_out.at[pl.ds(c * NP + r0, EC)])


def _sc_deg_body(dstp, out, dst_v, hist):
    c = lax.axis_index("c")
    s = lax.axis_index("s")
    wid = c * NS + s

    def zb(i, _):
        hist[pl.ds(i * 16, 16)] = jnp.zeros((16,), _F32)
        return 0
    lax.fori_loop(0, HN // 16, zb, 0)

    ones16 = jnp.ones((16,), _F32)

    def chunk(j, _):
        base = wid * DTILE + j * DC
        pltpu.sync_copy(dstp.at[pl.ds(base, DC)], dst_v)
        for t in range(DC // 16):
            idx = dst_v[pl.ds(t * 16, 16)]
            plsc.addupdate_scatter(hist, [idx], ones16)
        return 0
    lax.fori_loop(0, NDCHUNK, chunk, 0)

    pltpu.sync_copy(hist, out.at[pl.ds(wid * HN, HN)])


@functools.lru_cache(maxsize=None)
def _build_sc_kernels():
    mesh = plsc.VectorSubcoreMesh(core_axis_name="c", subcore_axis_name="s")
    agg = pl.kernel(
        _sc_agg_body,
        out_type=jax.ShapeDtypeStruct((NC * NP, HD), _F32),
        mesh=mesh,
        scratch_types=(
            pltpu.VMEM((CPH, EC), jnp.int32),
            pltpu.VMEM((CPH, EC), jnp.int32),
            pltpu.VMEM((EC, HD), _F32),
            pltpu.VMEM((EC, HD), _F32),
            pltpu.VMEM_SHARED((NP, HD), _F32),
            pltpu.SemaphoreType.DMA,
            pltpu.SemaphoreType.DMA,
        ),
    )
    deg = pl.kernel(
        _sc_deg_body,
        out_type=jax.ShapeDtypeStruct((NC * NS * HN,), _F32),
        mesh=mesh,
        compiler_params=pltpu.CompilerParams(needs_layout_passes=False),
        scratch_types=(
            pltpu.VMEM((DC,), jnp.int32),
            pltpu.VMEM((HN,), _F32),
        ),
    )
    return agg, deg


def _pad_edges(src, dst):
    pad = EPAD - E
    srcp = jnp.concatenate([src, jnp.zeros((pad,), jnp.int32)])
    dstp = jnp.concatenate([dst, jnp.full((pad,), NP - 1, jnp.int32)])
    return (srcp.reshape(NS, NCHUNK, EC), dstp.reshape(NS, NCHUNK, EC),
            dstp)


def _sc_agg(hpair, src3, dst3):
    agg = _build_sc_kernels()[0](hpair.reshape(NC * N, HD), src3, dst3)
    return agg.reshape(NC, NP, HD)


def _sc_deg(dstp):
    out = _build_sc_kernels()[1](dstp)
    return out.reshape(NC * NS, HN)[:, :N].T


def _mask_body(x_ref, kb_ref, out_ref):
    xb = x_ref[...] * kb_ref[...]
    out_ref[0] = xb[:, :HD]
    out_ref[1] = xb[:, HD:]


def _mask_mul(x, kb):
    return pl.pallas_call(
        _mask_body,
        grid=(GRID,),
        in_specs=[pl.BlockSpec((BLK, D), lambda i: (i, 0)),
                  pl.BlockSpec((BLK, D), lambda i: (i, 0))],
        out_specs=pl.BlockSpec((NC, BLK, HD), lambda i: (0, i, 0)),
        out_shape=jax.ShapeDtypeStruct((NC, N, HD), _F32),
    )(x, kb)


def _dense_body(relu, agg_ref, deg_ref, h_ref, wl_ref, bl_ref, wr_ref,
                out_ref):
    aggc = jnp.concatenate([agg_ref[0], agg_ref[1]], axis=1)
    deg = jnp.sum(deg_ref[...], axis=1)[:, None]
    inv = 1.0 / jnp.maximum(deg, 1.0)
    hc = jnp.concatenate([h_ref[0], h_ref[1]], axis=1)
    out = (jnp.dot(aggc * inv, wl_ref[...].T,
                   preferred_element_type=_F32)
           + bl_ref[...]
           + jnp.dot(hc, wr_ref[...].T, preferred_element_type=_F32))
    if relu:
        out = jnp.maximum(out, 0.0)
    out_ref[0] = out[:, :HD]
    out_ref[1] = out[:, HD:]


def _dense(agg, deg, h, wl, bl, wr, relu):
    return pl.pallas_call(
        functools.partial(_dense_body, relu),
        grid=(GRID,),
        in_specs=[pl.BlockSpec((NC, BLK, HD), lambda i: (0, i, 0)),
                  pl.BlockSpec((BLK, NC * NS), lambda i: (i, 0)),
                  pl.BlockSpec((NC, BLK, HD), lambda i: (0, i, 0)),
                  pl.BlockSpec((HID, HID), lambda i: (0, 0)),
                  pl.BlockSpec((1, HID), lambda i: (0, 0)),
                  pl.BlockSpec((HID, HID), lambda i: (0, 0))],
        out_specs=pl.BlockSpec((NC, BLK, HD), lambda i: (0, i, 0)),
        out_shape=jax.ShapeDtypeStruct((NC, N, HD), _F32),
    )(agg, deg, h, wl, bl, wr)


def _pool_body(agg_ref, deg_ref, h_ref, wl_ref, bl_ref, wr_ref, batch_ref,
               gs_ref, cs_ref):
    aggc = jnp.concatenate([agg_ref[0], agg_ref[1]], axis=1)
    deg = jnp.sum(deg_ref[...], axis=1)[:, None]
    inv = 1.0 / jnp.maximum(deg, 1.0)
    hc = jnp.concatenate([h_ref[0], h_ref[1]], axis=1)
    h3 = (jnp.dot(aggc * inv, wl_ref[...].T, preferred_element_type=_F32)
          + bl_ref[...]
          + jnp.dot(hc, wr_ref[...].T, preferred_element_type=_F32))
    bb = batch_ref[0, 0, :]
    gid = lax.broadcasted_iota(jnp.int32, (B, BLK), 0)
    onehot = (gid == bb[None, :]).astype(_F32)
    pool = jnp.dot(onehot, h3, preferred_element_type=_F32)
    cpool = jnp.dot(onehot, jnp.ones((BLK, 16), _F32),
                    preferred_element_type=_F32)

    @pl.when(pl.program_id(0) == 0)
    def _():
        gs_ref[...] = jnp.zeros_like(gs_ref)
        cs_ref[...] = jnp.zeros_like(cs_ref)

    gs_ref[...] += pool
    cs_ref[...] += cpool


def _pool(agg, deg, h, wl, bl, wr, batch3d):
    return pl.pallas_call(
        _pool_body,
        grid=(GRID,),
        in_specs=[pl.BlockSpec((NC, BLK, HD), lambda i: (0, i, 0)),
                  pl.BlockSpec((BLK, NC * NS), lambda i: (i, 0)),
                  pl.BlockSpec((NC, BLK, HD), lambda i: (0, i, 0)),
                  pl.BlockSpec((HID, HID), lambda i: (0, 0)),
                  pl.BlockSpec((1, HID), lambda i: (0, 0)),
                  pl.BlockSpec((HID, HID), lambda i: (0, 0)),
                  pl.BlockSpec((1, 1, BLK), lambda i: (i, 0, 0))],
        out_specs=[pl.BlockSpec((B, HID), lambda i: (0, 0)),
                   pl.BlockSpec((B, 16), lambda i: (0, 0))],
        out_shape=[jax.ShapeDtypeStruct((B, HID), _F32),
                   jax.ShapeDtypeStruct((B, 16), _F32)],
    )(agg, deg, h, wl, bl, wr, batch3d)


def _graph_chain(gs, cs, wvp, bvp, wvi, bvi, wo, bo, lng, lnb,
                 wd1, bd1, wd2, bd2):
    gvec = gs * (1.0 / jnp.maximum(cs[:, 0:1], 1.0))
    vv = jnp.dot(gvec, wvp.T, preferred_element_type=_F32) + bvp
    vi = jnp.dot(vv, wvi.T, preferred_element_type=_F32) + bvi
    att = jnp.dot(vi, wo.T, preferred_element_type=_F32) + bo
    m = jnp.mean(att, axis=-1, keepdims=True)
    cen = att - m
    var = jnp.mean(cen * cen, axis=-1, keepdims=True)
    ln = cen / jnp.sqrt(var + 1e-5) * lng + lnb
    d1 = jnp.maximum(jnp.dot(ln, wd1.T, preferred_element_type=_F32) + bd1,
                     0.0)
    pg = jnp.dot(d1, wd2.T, preferred_element_type=_F32) + bd2
    return gvec, pg


def _graph_body(gsa_ref, csa_ref, gsb_ref, csb_ref, lab_ref,
                wvp_ref, bvp_ref, wvi_ref, bvi_ref, wo_ref, bo_ref,
                lng_ref, lnb_ref, wd1_ref, bd1_ref, wd2_ref, bd2_ref,
                pga_ref, pgb_ref, lc_ref):
    args = (wvp_ref[...], bvp_ref[...], wvi_ref[...], bvi_ref[...],
            wo_ref[...], bo_ref[...], lng_ref[...], lnb_ref[...],
            wd1_ref[...], bd1_ref[...], wd2_ref[...], bd2_ref[...])
    gva, pga = _graph_chain(gsa_ref[...], csa_ref[...], *args)
    gvb, pgb = _graph_chain(gsb_ref[...], csb_ref[...], *args)
    pga_ref[...] = pga
    pgb_ref[...] = pgb
    diff = gva - gvb + 1e-6
    dist = jnp.sqrt(jnp.sum(diff * diff, axis=-1, keepdims=True))
    lab = lab_ref[...]
    hinge = jnp.maximum(MARGIN - dist, 0.0)
    lc = jnp.mean(lab * dist * dist + (1.0 - lab) * hinge * hinge)
    lc_ref[...] = jnp.full((1, 1), lc, _F32)


def _graph(gsa, csa, gsb, csb, lab2d, wvp, bvp, wvi, bvi, wo, bo,
           lng, lnb, wd1, bd1, wd2, bd2):
    return pl.pallas_call(
        _graph_body,
        out_shape=[jax.ShapeDtypeStruct((B, HID), _F32),
                   jax.ShapeDtypeStruct((B, HID), _F32),
                   jax.ShapeDtypeStruct((1, 1), _F32)],
    )(gsa, csa, gsb, csb, lab2d, wvp, bvp, wvi, bvi, wo, bo,
      lng, lnb, wd1, bd1, wd2, bd2)


def _lpred_body(x_ref, mi_ref, batch_ref, pg_ref, out_ref, cnt_ref):
    bb = batch_ref[0, 0, :]
    nid = lax.broadcasted_iota(jnp.int32, (BLK, B), 1)
    oh = (nid == bb[:, None]).astype(_F32)
    pgb = jnp.dot(oh, pg_ref[...], preferred_element_type=_F32)
    d = pgb - x_ref[...]
    mi = mi_ref[...]
    val = jnp.sum(d * d * mi)
    cval = jnp.sum(mi)

    @pl.when(pl.program_id(0) == 0)
    def _():
        out_ref[...] = jnp.zeros((1, 1), _F32)
        cnt_ref[...] = jnp.zeros((1, 1), _F32)

    out_ref[...] = out_ref[...] + val
    cnt_ref[...] = cnt_ref[...] + cval


def _lpred(x, mi, batch3d, pg):
    return pl.pallas_call(
        _lpred_body,
        grid=(GRID,),
        in_specs=[pl.BlockSpec((BLK, D), lambda i: (i, 0)),
                  pl.BlockSpec((BLK, D), lambda i: (i, 0)),
                  pl.BlockSpec((1, 1, BLK), lambda i: (i, 0, 0)),
                  pl.BlockSpec((B, HID), lambda i: (0, 0))],
        out_specs=[pl.BlockSpec((1, 1), lambda i: (0, 0)),
                   pl.BlockSpec((1, 1), lambda i: (0, 0))],
        out_shape=[jax.ShapeDtypeStruct((1, 1), _F32),
                   jax.ShapeDtypeStruct((1, 1), _F32)],
    )(x, mi, batch3d, pg)


def _branch_nodes(x, src, dst, batch3d, kb, W1l, b1l, W1r, W2l, b2l, W2r,
                  W3l, b3l, W3r):
    src3, dst3, dstp = _pad_edges(src, dst)
    xbb = _mask_mul(x, kb)
    deg = _sc_deg(dstp)
    agg1 = _sc_agg(xbb, src3, dst3)
    h1 = _dense(agg1, deg, xbb, W1l, b1l, W1r, relu=True)
    agg2 = _sc_agg(h1, src3, dst3)
    h2 = _dense(agg2, deg, h1, W2l, b2l, W2r, relu=True)
    agg3 = _sc_agg(h2, src3, dst3)
    gs, cs = _pool(agg3, deg, h2, W3l, b3l, W3r, batch3d)
    return gs, cs


def kernel(x_a, edge_index_a, batch_a, x_b, edge_index_b, batch_b, labels,
           W1l, b1l, W1r, W2l, b2l, W2r, W3l, b3l, W3r, Wqp, bqp, Wkp, bkp,
           Wvp, bvp, Wqi, bqi, Wki, bki, Wvi, bvi, Wo, bo, ln_g, ln_b,
           Wd1, bd1, Wd2, bd2):
    src_a = edge_index_a[0]
    dst_a = edge_index_a[1]
    src_b = edge_index_b[0]
    dst_b = edge_index_b[1]
    batch3d_a = batch_a.reshape(GRID, 1, BLK)
    batch3d_b = batch_b.reshape(GRID, 1, BLK)
    b1l2 = b1l.reshape(1, HID)
    b2l2 = b2l.reshape(1, HID)
    b3l2 = b3l.reshape(1, HID)

    (kba, mia), (kbb, mib) = _make_masks()
    gs_a, cs_a = _branch_nodes(x_a, src_a, dst_a, batch3d_a, kba,
                               W1l, b1l2, W1r, W2l, b2l2, W2r,
                               W3l, b3l2, W3r)
    gs_b, cs_b = _branch_nodes(x_b, src_b, dst_b, batch3d_b, kbb,
                               W1l, b1l2, W1r, W2l, b2l2, W2r,
                               W3l, b3l2, W3r)

    pga, pgb, lc = _graph(gs_a, cs_a, gs_b, cs_b, labels.reshape(B, 1),
                          Wvp, bvp.reshape(1, HID), Wvi, bvi.reshape(1, HID),
                          Wo, bo.reshape(1, HID), ln_g.reshape(1, HID),
                          ln_b.reshape(1, HID), Wd1, bd1.reshape(1, HID),
                          Wd2, bd2.reshape(1, HID))

    sa, ca = _lpred(x_a, mia, batch3d_a, pga)
    sb, cb = _lpred(x_b, mib, batch3d_b, pgb)

    loss_con = lc[0, 0]
    lpa = sa[0, 0] / jnp.maximum(ca[0, 0], 1.0)
    lpb = sb[0, 0] / jnp.maximum(cb[0, 0], 1.0)
    loss_pred = (lpa + lpb) * 0.5
    total = ALPHA * loss_con + BETA * loss_pred
    return (total, loss_con, loss_pred)
